# baseline (speedup 1.0000x reference)
import jax
import jax.numpy as jnp
from jax import lax
from jax.experimental import pallas as pl
from jax.experimental.pallas import tpu as pltpu
from jax.experimental.pallas import tpu_sc as plsc

L = 16
NSTEP = 64
NSEG = 64
SEGW = L * NSTEP
NQ = SEGW * NSEG
B = 128
BPW = 4
TOPK = 5
NCHUNK = 8
CH = NQ // NCHUNK
SEG_PER_CHUNK = NSEG // NCHUNK
NEG_INF = float("-inf")
BIG = 2 ** 30

_GDN = lax.GatherDimensionNumbers(
    offset_dims=(), collapsed_slice_dims=(0,), start_index_map=(0,))


def _lane_perm(v, perm):
    return lax.gather(v, perm[:, None], dimension_numbers=_GDN,
                      slice_sizes=(1,),
                      mode=lax.GatherScatterMode.PROMISE_IN_BOUNDS)


def _bcast_reduce(v, op):
    for sh in (8, 4, 2, 1):
        perm = jnp.bitwise_xor(lax.iota(jnp.int32, L), sh)
        v = op(v, _lane_perm(v, perm))
    return v


def _tec_body(cam_ref, hw_ref, out_ref, data, cval, hwbuf, rowbuf, sem):
    nc = 2
    wid = lax.axis_index("s") * nc + lax.axis_index("c")
    iota = lax.iota(jnp.int32, L)
    minf = jnp.full((L,), NEG_INF, jnp.float32)
    bigv = jnp.full((L,), BIG, jnp.int32)

    pltpu.sync_copy(hw_ref, hwbuf)
    hv = hwbuf[pl.ds(0, L)]
    wv = hwbuf[pl.ds(L, L)]

    def seg_vmax(sbase):
        accs = [minf] * 4
        for j in range(NSTEP):
            v = data[pl.ds(sbase + j * L, L)]
            a = j // (NSTEP // 4)
            accs[a] = jnp.maximum(accs[a], v)
        return jnp.maximum(jnp.maximum(accs[0], accs[1]),
                           jnp.maximum(accs[2], accs[3]))

    def one_batch(bb, _):
        b = wid * BPW + bb
        src = cam_ref.at[b, 1]
        cps = [None] * NCHUNK
        cps[0] = pltpu.async_copy(src.at[pl.ds(0, CH)],
                                  data.at[pl.ds(0, CH)], sem)
        for c in range(NCHUNK):
            if c + 1 < NCHUNK:
                cps[c + 1] = pltpu.async_copy(
                    src.at[pl.ds((c + 1) * CH, CH)],
                    data.at[pl.ds((c + 1) * CH, CH)], sem)
            cps[c].wait()

            def level1(s, _c):
                cval[pl.ds(s * L, L)] = seg_vmax(s * SEGW)
                return 0

            lax.fori_loop(c * SEG_PER_CHUNK, (c + 1) * SEG_PER_CHUNK,
                          level1, 0)

        ridx = jnp.zeros((L,), jnp.int32)
        for t in range(TOPK):
            def scanv(s, m):
                return jnp.maximum(m, cval[pl.ds(s * L, L)])

            mxv = _bcast_reduce(lax.fori_loop(0, NSEG, scanv, minf),
                                jnp.maximum)

            def scanseg(s, best):
                hit = cval[pl.ds(s * L, L)] == mxv
                return jnp.minimum(best, jnp.where(hit, s, BIG))

            segv = _bcast_reduce(lax.fori_loop(0, NSEG, scanseg, bigv),
                                 jnp.minimum)
            sbase = pl.multiple_of(segv[0] * SEGW, SEGW)

            def rescan(j, imin):
                off = sbase + j * L
                v = data[pl.ds(off, L)]
                cand = jnp.where(v == mxv, iota + off, BIG)
                return jnp.minimum(imin, cand)

            rvec = _bcast_reduce(lax.fori_loop(0, NSTEP, rescan, bigv),
                                 jnp.minimum)
            ridx = jnp.where(iota == t, rvec, ridx)
            if t < TOPK - 1:
                r = rvec[0]
                wbase = pl.multiple_of(jnp.right_shift(r, 4) * L, L)
                win = data[pl.ds(wbase, L)]
                lane_r = jnp.bitwise_and(r, L - 1)
                data[pl.ds(wbase, L)] = jnp.where(iota == lane_r, minf, win)
                cbase = pl.multiple_of(jnp.right_shift(r, 10) * L, L)
                cval[pl.ds(cbase, L)] = seg_vmax(sbase)

        y = jnp.right_shift(ridx, 8)
        x = jnp.bitwise_and(ridx, 255)
        sx = jnp.right_shift(x * wv, 8)
        sy = jnp.right_shift(y * hv, 8)
        half = jnp.right_shift(iota, 1)
        rowv = jnp.where(jnp.bitwise_and(iota, 1) == 0,
                         _lane_perm(sx, half), _lane_perm(sy, half))
        rowbuf[...] = jnp.where(iota < 2 * TOPK, rowv, 0)
        pltpu.sync_copy(rowbuf, out_ref.at[b])
        return 0

    lax.fori_loop(0, BPW, one_batch, 0)


@jax.jit
def kernel(cam, features_shape):
    fs = features_shape.astype(jnp.int32)
    hw = jnp.concatenate([jnp.full((L,), fs[2]), jnp.full((L,), fs[3])])
    mesh = plsc.VectorSubcoreMesh(core_axis_name="c", subcore_axis_name="s")
    run = pl.kernel(
        _tec_body,
        out_type=jax.ShapeDtypeStruct((B, L), jnp.int32),
        mesh=mesh,
        compiler_params=pltpu.CompilerParams(use_tc_tiling_on_sc=True),
        scratch_types=[
            pltpu.VMEM((NQ,), jnp.float32),
            pltpu.VMEM((NSEG * L,), jnp.float32),
            pltpu.VMEM((2 * L,), jnp.int32),
            pltpu.VMEM((L,), jnp.int32),
            pltpu.SemaphoreType.DMA,
        ],
    )
    out16 = run(cam, hw)
    return out16[:, :2 * TOPK].reshape(B, TOPK, 2)


# speedup vs baseline: 1.3061x; 1.3061x over previous
import jax
import jax.numpy as jnp
from jax import lax
from jax.experimental import pallas as pl
from jax.experimental.pallas import tpu as pltpu
from jax.experimental.pallas import tpu_sc as plsc

L = 16
NSTEP = 64
NSEG = 64
SEGW = L * NSTEP
NQ = SEGW * NSEG
B = 128
BPW = 4
TOPK = 5
NEG_INF = float("-inf")
BIG = 2 ** 30

_GDN = lax.GatherDimensionNumbers(
    offset_dims=(), collapsed_slice_dims=(0,), start_index_map=(0,))


def _lane_perm(v, perm):
    return lax.gather(v, perm[:, None], dimension_numbers=_GDN,
                      slice_sizes=(1,),
                      mode=lax.GatherScatterMode.PROMISE_IN_BOUNDS)


def _bcast_reduce(v, op):
    for sh in (8, 4, 2, 1):
        perm = jnp.bitwise_xor(lax.iota(jnp.int32, L), sh)
        v = op(v, _lane_perm(v, perm))
    return v


NCHUNK = 4
CH = NQ // NCHUNK
SEG_PER_CHUNK = NSEG // NCHUNK


def _tec_body(cam_ref, hw_ref, out_ref, data, cval, cidx, hwbuf, rowbuf, sem):
    nc = 2
    wid = lax.axis_index("s") * nc + lax.axis_index("c")
    iota = lax.iota(jnp.int32, L)
    minf = jnp.full((L,), NEG_INF, jnp.float32)
    zeros_i = jnp.zeros((L,), jnp.int32)

    pltpu.sync_copy(hw_ref, hwbuf)
    hv = hwbuf[pl.ds(0, L)]
    wv = hwbuf[pl.ds(L, L)]

    def merge(p, q):
        (ma, ia), (mb, ib) = p, q
        c = mb > ma
        return (jnp.where(c, mb, ma), jnp.where(c, ib, ia))

    def seg_maxes(sbase):
        accs = [(minf, zeros_i) for _ in range(4)]
        for j in range(NSTEP):
            off = sbase + j * L
            v = data[pl.ds(off, L)]
            iv = iota + off
            a = j // (NSTEP // 4)
            m, mi = accs[a]
            c = v > m
            accs[a] = (jnp.where(c, v, m), jnp.where(c, iv, mi))
        return merge(merge(accs[0], accs[1]), merge(accs[2], accs[3]))

    def one_batch(bb, _):
        b = wid * BPW + bb
        src = cam_ref.at[b, 1]
        cps = [None] * NCHUNK
        cps[0] = pltpu.async_copy(src.at[pl.ds(0, CH)],
                                  data.at[pl.ds(0, CH)], sem)

        def level1(s, _c):
            m, mi = seg_maxes(s * SEGW)
            cval[pl.ds(s * L, L)] = m
            cidx[pl.ds(s * L, L)] = mi
            return 0

        for c in range(NCHUNK):
            if c + 1 < NCHUNK:
                cps[c + 1] = pltpu.async_copy(
                    src.at[pl.ds((c + 1) * CH, CH)],
                    data.at[pl.ds((c + 1) * CH, CH)], sem)
            cps[c].wait()
            lax.fori_loop(c * SEG_PER_CHUNK, (c + 1) * SEG_PER_CHUNK,
                          level1, 0)

        ridx = zeros_i
        for t in range(TOPK):
            def scan(s, carry):
                m, mi = carry
                v = cval[pl.ds(s * L, L)]
                vi = cidx[pl.ds(s * L, L)]
                c = v > m
                return (jnp.where(c, v, m), jnp.where(c, vi, mi))

            m, mi = lax.fori_loop(0, NSEG, scan, (minf, zeros_i))
            mx = _bcast_reduce(m, jnp.maximum)
            rvec = _bcast_reduce(jnp.where(m == mx, mi, BIG), jnp.minimum)
            ridx = jnp.where(iota == t, rvec, ridx)
            if t < TOPK - 1:
                r = rvec[0]
                wbase = pl.multiple_of(jnp.right_shift(r, 4) * L, L)
                win = data[pl.ds(wbase, L)]
                lane_r = jnp.bitwise_and(r, L - 1)
                data[pl.ds(wbase, L)] = jnp.where(iota == lane_r, minf, win)
                seg = jnp.right_shift(r, 10)
                m2, mi2 = seg_maxes(pl.multiple_of(seg * SEGW, SEGW))
                cbase = pl.multiple_of(seg * L, L)
                cval[pl.ds(cbase, L)] = m2
                cidx[pl.ds(cbase, L)] = mi2

        y = jnp.right_shift(ridx, 8)
        x = jnp.bitwise_and(ridx, 255)
        sx = jnp.right_shift(x * wv, 8)
        sy = jnp.right_shift(y * hv, 8)
        half = jnp.right_shift(iota, 1)
        rowv = jnp.where(jnp.bitwise_and(iota, 1) == 0,
                         _lane_perm(sx, half), _lane_perm(sy, half))
        rowbuf[...] = jnp.where(iota < 2 * TOPK, rowv, 0)
        pltpu.sync_copy(rowbuf, out_ref.at[b])
        return 0

    lax.fori_loop(0, BPW, one_batch, 0)


@jax.jit
def kernel(cam, features_shape):
    fs = features_shape.astype(jnp.int32)
    hw = jnp.concatenate([jnp.full((L,), fs[2]), jnp.full((L,), fs[3])])
    mesh = plsc.VectorSubcoreMesh(core_axis_name="c", subcore_axis_name="s")
    run = pl.kernel(
        _tec_body,
        out_type=jax.ShapeDtypeStruct((B, L), jnp.int32),
        mesh=mesh,
        compiler_params=pltpu.CompilerParams(use_tc_tiling_on_sc=True),
        scratch_types=[
            pltpu.VMEM((NQ,), jnp.float32),
            pltpu.VMEM((NSEG * L,), jnp.float32),
            pltpu.VMEM((NSEG * L,), jnp.int32),
            pltpu.VMEM((2 * L,), jnp.int32),
            pltpu.VMEM((L,), jnp.int32),
            pltpu.SemaphoreType.DMA,
        ],
    )
    out16 = run(cam, hw)
    return out16[:, :2 * TOPK].reshape(B, TOPK, 2)
